# baseline (speedup 1.0000x reference)
import functools

import jax
import jax.numpy as jnp
from jax import lax
from jax.experimental import pallas as pl
from jax.experimental.pallas import tpu as pltpu
from jax.experimental.pallas import tpu_sc as plsc

B = 4096
L = 200
EMBED = 64
EMBED_P = 128
HIDDEN = 128
CLASSES = 1000

NC = 2
NS = 16
NW = NC * NS
B_PER_W = B // NW
LANES = 16
NCH = EMBED // LANES

CHUNKS = ((0, 104), (104, 96))

_mesh = plsc.VectorSubcoreMesh(core_axis_name="c", subcore_axis_name="s")


@functools.partial(
    pl.kernel,
    out_type=jax.ShapeDtypeStruct((B, EMBED), jnp.float32),
    mesh=_mesh,
    scratch_types=[
        pltpu.VMEM((B_PER_W * L,), jnp.int32),
        pltpu.VMEM((L, EMBED_P), jnp.float32),
        pltpu.VMEM((L, EMBED_P), jnp.float32),
        pltpu.VMEM((B_PER_W, EMBED), jnp.float32),
        pltpu.SemaphoreType.DMA,
        pltpu.SemaphoreType.DMA,
    ],
)
def _pool_kernel(table_hbm, text_hbm, out_hbm, idx_v, buf_a, buf_b, pooled_v,
                 sem_a, sem_b):
    wid = lax.axis_index("s") * NC + lax.axis_index("c")
    base = wid * B_PER_W
    pltpu.sync_copy(text_hbm.at[pl.ds(base * L, B_PER_W * L)], idx_v)

    def issue(r, buf, sem):
        rbase = pl.multiple_of(r * L, 8)
        for off, size in CHUNKS:
            pltpu.async_copy(
                table_hbm.at[idx_v.at[pl.ds(rbase + off, size)]],
                buf.at[pl.ds(off, size), :],
                sem,
            )

    def drain(sem):
        pltpu.make_async_copy(
            table_hbm.at[pl.ds(0, L), :], buf_a, sem
        ).wait()

    def accumulate(r, buf):
        def acc_body(t, accs):
            return tuple(
                accs[c] + buf[t, pl.ds(c * LANES, LANES)] for c in range(NCH)
            )

        accs = tuple(jnp.zeros((LANES,), jnp.float32) for _ in range(NCH))
        accs = lax.fori_loop(0, L, acc_body, accs, unroll=8)
        scale = jnp.float32(1.0 / L)
        for c in range(NCH):
            pooled_v[r, pl.ds(c * LANES, LANES)] = accs[c] * scale

    issue(0, buf_a, sem_a)

    def pair_body(k, carry):
        ra = 2 * k
        rb = 2 * k + 1
        issue(rb, buf_b, sem_b)
        drain(sem_a)
        accumulate(ra, buf_a)

        @pl.when(k < B_PER_W // 2 - 1)
        def _():
            issue(ra + 2, buf_a, sem_a)

        drain(sem_b)
        accumulate(rb, buf_b)
        return carry

    lax.fori_loop(0, B_PER_W // 2, pair_body, 0)
    pltpu.sync_copy(pooled_v, out_hbm.at[pl.ds(base, B_PER_W), :])


VOCAB = 1000000
TCOLS = 2048


def _transpose_pad_body(xt_ref, o_ref):
    eye = jnp.eye(EMBED, dtype=jnp.float32)
    t = lax.dot_general(
        xt_ref[...], eye, (((0,), (0,)), ((), ())),
        preferred_element_type=jnp.float32,
    )
    o_ref[:, :EMBED] = t
    o_ref[:, EMBED:] = jnp.zeros((TCOLS, EMBED_P - EMBED), jnp.float32)


def _transpose_pad(table_t):
    return pl.pallas_call(
        _transpose_pad_body,
        grid=(pl.cdiv(VOCAB, TCOLS),),
        in_specs=[pl.BlockSpec((EMBED, TCOLS), lambda i: (0, i))],
        out_specs=pl.BlockSpec((TCOLS, EMBED_P), lambda i: (i, 0)),
        out_shape=jax.ShapeDtypeStruct((VOCAB, EMBED_P), jnp.float32),
    )(table_t)


def _mlp_body(x_ref, fcw_ref, fcb_ref, outw_ref, outb_ref, o_ref):
    h = jnp.tanh(
        lax.dot_general(
            x_ref[...], fcw_ref[...], (((1,), (1,)), ((), ())),
            preferred_element_type=jnp.float32,
        )
        + fcb_ref[...]
    )
    o_ref[...] = (
        lax.dot_general(
            h, outw_ref[...], (((1,), (1,)), ((), ())),
            preferred_element_type=jnp.float32,
        )
        + outb_ref[...]
    )


def kernel(text, embed_table, fc_w, fc_b, out_w, out_b):
    table_p = _transpose_pad(embed_table.T)
    pooled = _pool_kernel(table_p, text.reshape(-1))

    bt = 512
    out = pl.pallas_call(
        _mlp_body,
        grid=(B // bt,),
        in_specs=[
            pl.BlockSpec((bt, EMBED), lambda i: (i, 0)),
            pl.BlockSpec((HIDDEN, EMBED), lambda i: (0, 0)),
            pl.BlockSpec((1, HIDDEN), lambda i: (0, 0)),
            pl.BlockSpec((CLASSES, HIDDEN), lambda i: (0, 0)),
            pl.BlockSpec((1, CLASSES), lambda i: (0, 0)),
        ],
        out_specs=pl.BlockSpec((bt, CLASSES), lambda i: (i, 0)),
        out_shape=jax.ShapeDtypeStruct((B, CLASSES), jnp.float32),
    )(pooled, fc_w, fc_b.reshape(1, HIDDEN), out_w, out_b.reshape(1, CLASSES))
    return out


# speedup vs baseline: 1.0318x; 1.0318x over previous
import functools

import jax
import jax.numpy as jnp
from jax import lax
from jax.experimental import pallas as pl
from jax.experimental.pallas import tpu as pltpu
from jax.experimental.pallas import tpu_sc as plsc

B = 4096
L = 200
EMBED = 64
EMBED_P = 128
HIDDEN = 128
CLASSES = 1000

NC = 2
NS = 16
NW = NC * NS
B_PER_W = B // NW
LANES = 16
NCH = EMBED // LANES

CHUNKS = ((0, 104), (104, 96))

_mesh = plsc.VectorSubcoreMesh(core_axis_name="c", subcore_axis_name="s")


@functools.partial(
    pl.kernel,
    out_type=jax.ShapeDtypeStruct((B, EMBED), jnp.float32),
    mesh=_mesh,
    scratch_types=[
        pltpu.VMEM((B_PER_W * L,), jnp.int32),
        pltpu.VMEM((L, EMBED_P), jnp.float32),
        pltpu.VMEM((L, EMBED_P), jnp.float32),
        pltpu.VMEM((B_PER_W, EMBED), jnp.float32),
        pltpu.SemaphoreType.DMA,
        pltpu.SemaphoreType.DMA,
    ],
)
def _pool_kernel(table_hbm, text_hbm, out_hbm, idx_v, buf_a, buf_b, pooled_v,
                 sem_a, sem_b):
    wid = lax.axis_index("s") * NC + lax.axis_index("c")
    base = wid * B_PER_W
    pltpu.sync_copy(text_hbm.at[pl.ds(base * L, B_PER_W * L)], idx_v)

    def issue(r, buf, sem):
        rbase = pl.multiple_of(r * L, 8)
        for off, size in CHUNKS:
            pltpu.async_copy(
                table_hbm.at[idx_v.at[pl.ds(rbase + off, size)]],
                buf.at[pl.ds(off, size), :],
                sem,
            )

    def drain(sem):
        pltpu.make_async_copy(
            table_hbm.at[pl.ds(0, L), :], buf_a, sem
        ).wait()

    def accumulate(r, buf):
        def acc_body(t, accs):
            return tuple(
                accs[c] + buf[t, pl.ds(c * LANES, LANES)] for c in range(NCH)
            )

        accs = tuple(jnp.zeros((LANES,), jnp.float32) for _ in range(NCH))
        accs = lax.fori_loop(0, L, acc_body, accs, unroll=8)
        scale = jnp.float32(1.0 / L)
        for c in range(NCH):
            pooled_v[r, pl.ds(c * LANES, LANES)] = accs[c] * scale

    issue(0, buf_a, sem_a)

    def pair_body(k, carry):
        ra = 2 * k
        rb = 2 * k + 1
        issue(rb, buf_b, sem_b)
        drain(sem_a)
        accumulate(ra, buf_a)

        @pl.when(k < B_PER_W // 2 - 1)
        def _():
            issue(ra + 2, buf_a, sem_a)

        drain(sem_b)
        accumulate(rb, buf_b)
        return carry

    lax.fori_loop(0, B_PER_W // 2, pair_body, 0)
    pltpu.sync_copy(pooled_v, out_hbm.at[pl.ds(base, B_PER_W), :])


VOCAB = 1000000
TCOLS = 2048


def _transpose_pad_body(xt_ref, o_ref):
    o_ref[:, :EMBED] = xt_ref[...].T
    o_ref[:, EMBED:] = jnp.zeros((TCOLS, EMBED_P - EMBED), jnp.float32)


def _transpose_pad(table_t):
    return pl.pallas_call(
        _transpose_pad_body,
        grid=(pl.cdiv(VOCAB, TCOLS),),
        in_specs=[pl.BlockSpec((EMBED, TCOLS), lambda i: (0, i))],
        out_specs=pl.BlockSpec((TCOLS, EMBED_P), lambda i: (i, 0)),
        out_shape=jax.ShapeDtypeStruct((VOCAB, EMBED_P), jnp.float32),
    )(table_t)


def _mlp_body(x_ref, fcw_ref, fcb_ref, outw_ref, outb_ref, o_ref):
    h = jnp.tanh(
        lax.dot_general(
            x_ref[...], fcw_ref[...], (((1,), (1,)), ((), ())),
            preferred_element_type=jnp.float32,
        )
        + fcb_ref[...]
    )
    o_ref[...] = (
        lax.dot_general(
            h, outw_ref[...], (((1,), (1,)), ((), ())),
            preferred_element_type=jnp.float32,
        )
        + outb_ref[...]
    )


def kernel(text, embed_table, fc_w, fc_b, out_w, out_b):
    table_p = _transpose_pad(embed_table.T)
    pooled = _pool_kernel(table_p, text.reshape(-1))

    bt = 512
    out = pl.pallas_call(
        _mlp_body,
        grid=(B // bt,),
        in_specs=[
            pl.BlockSpec((bt, EMBED), lambda i: (i, 0)),
            pl.BlockSpec((HIDDEN, EMBED), lambda i: (0, 0)),
            pl.BlockSpec((1, HIDDEN), lambda i: (0, 0)),
            pl.BlockSpec((CLASSES, HIDDEN), lambda i: (0, 0)),
            pl.BlockSpec((1, CLASSES), lambda i: (0, 0)),
        ],
        out_specs=pl.BlockSpec((bt, CLASSES), lambda i: (i, 0)),
        out_shape=jax.ShapeDtypeStruct((B, CLASSES), jnp.float32),
    )(pooled, fc_w, fc_b.reshape(1, HIDDEN), out_w, out_b.reshape(1, CLASSES))
    return out


# speedup vs baseline: 1.2571x; 1.2184x over previous
import functools

import jax
import jax.numpy as jnp
from jax import lax
from jax.experimental import pallas as pl
from jax.experimental.pallas import tpu as pltpu
from jax.experimental.pallas import tpu_sc as plsc

B = 4096
L = 200
EMBED = 64
EMBED_P = 128
HIDDEN = 128
CLASSES = 1000

NC = 2
NS = 16
NW = NC * NS
B_PER_W = B // NW
LANES = 16
NCH = EMBED // LANES

CHUNKS = ((0, 104), (104, 96))

_mesh = plsc.VectorSubcoreMesh(core_axis_name="c", subcore_axis_name="s")


@functools.partial(
    pl.kernel,
    out_type=jax.ShapeDtypeStruct((B, EMBED), jnp.float32),
    mesh=_mesh,
    scratch_types=[
        pltpu.VMEM((B_PER_W * L,), jnp.int32),
        pltpu.VMEM((L, EMBED_P), jnp.float32),
        pltpu.VMEM((L, EMBED_P), jnp.float32),
        pltpu.VMEM((B_PER_W, EMBED), jnp.float32),
        pltpu.SemaphoreType.DMA,
        pltpu.SemaphoreType.DMA,
    ],
)
def _pool_kernel(table_hbm, text_hbm, out_hbm, idx_v, buf_a, buf_b, pooled_v,
                 sem_a, sem_b):
    wid = lax.axis_index("s") * NC + lax.axis_index("c")
    base = wid * B_PER_W
    pltpu.sync_copy(text_hbm.at[pl.ds(base * L, B_PER_W * L)], idx_v)

    def issue(r, buf, sem):
        rbase = pl.multiple_of(r * L, 8)
        for off, size in CHUNKS:
            pltpu.async_copy(
                table_hbm.at[idx_v.at[pl.ds(rbase + off, size)]],
                buf.at[pl.ds(off, size), :],
                sem,
            )

    def drain(sem):
        pltpu.make_async_copy(
            table_hbm.at[pl.ds(0, L), :], buf_a, sem
        ).wait()

    def accumulate(r, buf):
        def acc_body(t, accs):
            return tuple(
                accs[c] + buf[t, pl.ds(c * LANES, LANES)] for c in range(NCH)
            )

        accs = tuple(jnp.zeros((LANES,), jnp.float32) for _ in range(NCH))
        accs = lax.fori_loop(0, L, acc_body, accs, unroll=8)
        scale = jnp.float32(1.0 / L)
        for c in range(NCH):
            pooled_v[r, pl.ds(c * LANES, LANES)] = accs[c] * scale

    issue(0, buf_a, sem_a)

    def pair_body(k, carry):
        ra = 2 * k
        rb = 2 * k + 1
        issue(rb, buf_b, sem_b)
        drain(sem_a)
        accumulate(ra, buf_a)

        @pl.when(k < B_PER_W // 2 - 1)
        def _():
            issue(ra + 2, buf_a, sem_a)

        drain(sem_b)
        accumulate(rb, buf_b)
        return carry

    lax.fori_loop(0, B_PER_W // 2, pair_body, 0)
    pltpu.sync_copy(pooled_v, out_hbm.at[pl.ds(base, B_PER_W), :])


VOCAB = 1000000
TCOLS = 4096


def _transpose_pad_body(xt_ref, o_ref):
    o_ref[:, :EMBED] = xt_ref[...].T
    o_ref[:, EMBED:] = jnp.zeros((TCOLS, EMBED_P - EMBED), jnp.float32)


def _transpose_pad(table_t):
    return pl.pallas_call(
        _transpose_pad_body,
        grid=(pl.cdiv(VOCAB, TCOLS),),
        in_specs=[pl.BlockSpec((EMBED, TCOLS), lambda i: (0, i))],
        out_specs=pl.BlockSpec((TCOLS, EMBED_P), lambda i: (i, 0)),
        out_shape=jax.ShapeDtypeStruct((VOCAB, EMBED_P), jnp.float32),
    )(table_t)


def _mlp_body(x_ref, fcw_ref, fcb_ref, outw_ref, outb_ref, o_ref):
    h = jnp.tanh(
        lax.dot_general(
            x_ref[...], fcw_ref[...], (((1,), (1,)), ((), ())),
            preferred_element_type=jnp.float32,
        )
        + fcb_ref[...]
    )
    o_ref[...] = (
        lax.dot_general(
            h, outw_ref[...], (((1,), (1,)), ((), ())),
            preferred_element_type=jnp.float32,
        )
        + outb_ref[...]
    )


def kernel(text, embed_table, fc_w, fc_b, out_w, out_b):
    table_p = _transpose_pad(embed_table.T)
    pooled = _pool_kernel(table_p, text.reshape(-1))

    bt = 512
    out = pl.pallas_call(
        _mlp_body,
        grid=(B // bt,),
        in_specs=[
            pl.BlockSpec((bt, EMBED), lambda i: (i, 0)),
            pl.BlockSpec((HIDDEN, EMBED), lambda i: (0, 0)),
            pl.BlockSpec((1, HIDDEN), lambda i: (0, 0)),
            pl.BlockSpec((CLASSES, HIDDEN), lambda i: (0, 0)),
            pl.BlockSpec((1, CLASSES), lambda i: (0, 0)),
        ],
        out_specs=pl.BlockSpec((bt, CLASSES), lambda i: (i, 0)),
        out_shape=jax.ShapeDtypeStruct((B, CLASSES), jnp.float32),
    )(pooled, fc_w, fc_b.reshape(1, HIDDEN), out_w, out_b.reshape(1, CLASSES))
    return out


# speedup vs baseline: 1.4348x; 1.1414x over previous
import functools

import jax
import jax.numpy as jnp
from jax import lax
from jax.experimental import pallas as pl
from jax.experimental.pallas import tpu as pltpu
from jax.experimental.pallas import tpu_sc as plsc

B = 4096
L = 200
EMBED = 64
EMBED_P = 128
HIDDEN = 128
CLASSES = 1000

NC = 2
NS = 16
NW = NC * NS
B_PER_W = B // NW
LANES = 16
NCH = EMBED // LANES

CHUNKS = ((0, 104), (104, 96))

_mesh = plsc.VectorSubcoreMesh(core_axis_name="c", subcore_axis_name="s")


@functools.partial(
    pl.kernel,
    out_type=jax.ShapeDtypeStruct((B, EMBED), jnp.float32),
    mesh=_mesh,
    scratch_types=[
        pltpu.VMEM((B_PER_W * L,), jnp.int32),
        pltpu.VMEM((L, EMBED_P), jnp.float32),
        pltpu.VMEM((L, EMBED_P), jnp.float32),
        pltpu.VMEM((B_PER_W, EMBED), jnp.float32),
        pltpu.SemaphoreType.DMA,
        pltpu.SemaphoreType.DMA,
    ],
)
def _pool_kernel(table_hbm, text_hbm, out_hbm, idx_v, buf_a, buf_b, pooled_v,
                 sem_a, sem_b):
    wid = lax.axis_index("s") * NC + lax.axis_index("c")
    base = wid * B_PER_W
    pltpu.sync_copy(text_hbm.at[pl.ds(base * L, B_PER_W * L)], idx_v)

    def issue(r, buf, sem):
        rbase = pl.multiple_of(r * L, 8)
        for off, size in CHUNKS:
            pltpu.async_copy(
                table_hbm.at[idx_v.at[pl.ds(rbase + off, size)]],
                buf.at[pl.ds(off, size), :],
                sem,
            )

    def drain(sem):
        pltpu.make_async_copy(
            table_hbm.at[pl.ds(0, L), :], buf_a, sem
        ).wait()

    def accumulate(r, buf):
        def acc_body(t, accs):
            return tuple(
                accs[c] + buf[t, pl.ds(c * LANES, LANES)] for c in range(NCH)
            )

        accs = tuple(jnp.zeros((LANES,), jnp.float32) for _ in range(NCH))
        accs = lax.fori_loop(0, L, acc_body, accs, unroll=8)
        scale = jnp.float32(1.0 / L)
        for c in range(NCH):
            pooled_v[r, pl.ds(c * LANES, LANES)] = accs[c] * scale

    issue(0, buf_a, sem_a)

    def pair_body(k, carry):
        ra = 2 * k
        rb = 2 * k + 1
        issue(rb, buf_b, sem_b)
        drain(sem_a)
        accumulate(ra, buf_a)

        @pl.when(k < B_PER_W // 2 - 1)
        def _():
            issue(ra + 2, buf_a, sem_a)

        drain(sem_b)
        accumulate(rb, buf_b)
        return carry

    lax.fori_loop(0, B_PER_W // 2, pair_body, 0)
    pltpu.sync_copy(pooled_v, out_hbm.at[pl.ds(base, B_PER_W), :])


VOCAB = 1000000
TCOLS = 8192


def _transpose_pad_body(xt_ref, o_ref):
    o_ref[:, :EMBED] = xt_ref[...].T
    o_ref[:, EMBED:] = jnp.zeros((TCOLS, EMBED_P - EMBED), jnp.float32)


def _transpose_pad(table_t):
    return pl.pallas_call(
        _transpose_pad_body,
        grid=(pl.cdiv(VOCAB, TCOLS),),
        in_specs=[pl.BlockSpec((EMBED, TCOLS), lambda i: (0, i))],
        out_specs=pl.BlockSpec((TCOLS, EMBED_P), lambda i: (i, 0)),
        out_shape=jax.ShapeDtypeStruct((VOCAB, EMBED_P), jnp.float32),
    )(table_t)


def _mlp_body(x_ref, fcw_ref, fcb_ref, outw_ref, outb_ref, o_ref):
    h = jnp.tanh(
        lax.dot_general(
            x_ref[...], fcw_ref[...], (((1,), (1,)), ((), ())),
            preferred_element_type=jnp.float32,
        )
        + fcb_ref[...]
    )
    o_ref[...] = (
        lax.dot_general(
            h, outw_ref[...], (((1,), (1,)), ((), ())),
            preferred_element_type=jnp.float32,
        )
        + outb_ref[...]
    )


def kernel(text, embed_table, fc_w, fc_b, out_w, out_b):
    table_p = _transpose_pad(embed_table.T)
    pooled = _pool_kernel(table_p, text.reshape(-1))

    bt = 512
    out = pl.pallas_call(
        _mlp_body,
        grid=(B // bt,),
        in_specs=[
            pl.BlockSpec((bt, EMBED), lambda i: (i, 0)),
            pl.BlockSpec((HIDDEN, EMBED), lambda i: (0, 0)),
            pl.BlockSpec((1, HIDDEN), lambda i: (0, 0)),
            pl.BlockSpec((CLASSES, HIDDEN), lambda i: (0, 0)),
            pl.BlockSpec((1, CLASSES), lambda i: (0, 0)),
        ],
        out_specs=pl.BlockSpec((bt, CLASSES), lambda i: (i, 0)),
        out_shape=jax.ShapeDtypeStruct((B, CLASSES), jnp.float32),
    )(pooled, fc_w, fc_b.reshape(1, HIDDEN), out_w, out_b.reshape(1, CLASSES))
    return out


# speedup vs baseline: 1.4875x; 1.0367x over previous
import functools

import jax
import jax.numpy as jnp
from jax import lax
from jax.experimental import pallas as pl
from jax.experimental.pallas import tpu as pltpu
from jax.experimental.pallas import tpu_sc as plsc

B = 4096
L = 200
EMBED = 64
EMBED_P = 128
HIDDEN = 128
CLASSES = 1000

NC = 2
NS = 16
NW = NC * NS
B_PER_W = B // NW
LANES = 16
NCH = EMBED // LANES

CHUNKS = ((0, 104), (104, 96))

_mesh = plsc.VectorSubcoreMesh(core_axis_name="c", subcore_axis_name="s")


@functools.partial(
    pl.kernel,
    out_type=jax.ShapeDtypeStruct((B, EMBED), jnp.float32),
    mesh=_mesh,
    scratch_types=[
        pltpu.VMEM((B_PER_W * L,), jnp.int32),
        pltpu.VMEM((L, EMBED_P), jnp.float32),
        pltpu.VMEM((L, EMBED_P), jnp.float32),
        pltpu.VMEM((B_PER_W, EMBED), jnp.float32),
        pltpu.SemaphoreType.DMA,
        pltpu.SemaphoreType.DMA,
    ],
)
def _pool_kernel(table_hbm, text_hbm, out_hbm, idx_v, buf_a, buf_b, pooled_v,
                 sem_a, sem_b):
    wid = lax.axis_index("s") * NC + lax.axis_index("c")
    base = wid * B_PER_W
    pltpu.sync_copy(text_hbm.at[pl.ds(base * L, B_PER_W * L)], idx_v)

    def issue(r, buf, sem):
        rbase = pl.multiple_of(r * L, 8)
        for off, size in CHUNKS:
            pltpu.async_copy(
                table_hbm.at[idx_v.at[pl.ds(rbase + off, size)]],
                buf.at[pl.ds(off, size), :],
                sem,
            )

    def drain(sem):
        pltpu.make_async_copy(
            table_hbm.at[pl.ds(0, L), :], buf_a, sem
        ).wait()

    def accumulate(r, buf):
        def acc_body(t, accs):
            return tuple(
                accs[c] + buf[t, pl.ds(c * LANES, LANES)] for c in range(NCH)
            )

        accs = tuple(jnp.zeros((LANES,), jnp.float32) for _ in range(NCH))
        accs = lax.fori_loop(0, L, acc_body, accs, unroll=8)
        scale = jnp.float32(1.0 / L)
        for c in range(NCH):
            pooled_v[r, pl.ds(c * LANES, LANES)] = accs[c] * scale

    issue(0, buf_a, sem_a)

    def pair_body(k, carry):
        ra = 2 * k
        rb = 2 * k + 1
        issue(rb, buf_b, sem_b)
        drain(sem_a)
        accumulate(ra, buf_a)

        @pl.when(k < B_PER_W // 2 - 1)
        def _():
            issue(ra + 2, buf_a, sem_a)

        drain(sem_b)
        accumulate(rb, buf_b)
        return carry

    lax.fori_loop(0, B_PER_W // 2, pair_body, 0)
    pltpu.sync_copy(pooled_v, out_hbm.at[pl.ds(base, B_PER_W), :])


VOCAB = 1000000
TCOLS = 16384


def _transpose_pad_body(xt_ref, o_ref):
    o_ref[:, :EMBED] = xt_ref[...].T
    o_ref[:, EMBED:] = jnp.zeros((TCOLS, EMBED_P - EMBED), jnp.float32)


def _transpose_pad(table_t):
    return pl.pallas_call(
        _transpose_pad_body,
        grid=(pl.cdiv(VOCAB, TCOLS),),
        in_specs=[pl.BlockSpec((EMBED, TCOLS), lambda i: (0, i))],
        out_specs=pl.BlockSpec((TCOLS, EMBED_P), lambda i: (i, 0)),
        out_shape=jax.ShapeDtypeStruct((VOCAB, EMBED_P), jnp.float32),
    )(table_t)


def _mlp_body(x_ref, fcw_ref, fcb_ref, outw_ref, outb_ref, o_ref):
    h = jnp.tanh(
        lax.dot_general(
            x_ref[...], fcw_ref[...], (((1,), (1,)), ((), ())),
            preferred_element_type=jnp.float32,
        )
        + fcb_ref[...]
    )
    o_ref[...] = (
        lax.dot_general(
            h, outw_ref[...], (((1,), (1,)), ((), ())),
            preferred_element_type=jnp.float32,
        )
        + outb_ref[...]
    )


def kernel(text, embed_table, fc_w, fc_b, out_w, out_b):
    table_p = _transpose_pad(embed_table.T)
    pooled = _pool_kernel(table_p, text.reshape(-1))

    bt = 512
    out = pl.pallas_call(
        _mlp_body,
        grid=(B // bt,),
        in_specs=[
            pl.BlockSpec((bt, EMBED), lambda i: (i, 0)),
            pl.BlockSpec((HIDDEN, EMBED), lambda i: (0, 0)),
            pl.BlockSpec((1, HIDDEN), lambda i: (0, 0)),
            pl.BlockSpec((CLASSES, HIDDEN), lambda i: (0, 0)),
            pl.BlockSpec((1, CLASSES), lambda i: (0, 0)),
        ],
        out_specs=pl.BlockSpec((bt, CLASSES), lambda i: (i, 0)),
        out_shape=jax.ShapeDtypeStruct((B, CLASSES), jnp.float32),
    )(pooled, fc_w, fc_b.reshape(1, HIDDEN), out_w, out_b.reshape(1, CLASSES))
    return out


# speedup vs baseline: 1.5449x; 1.0386x over previous
import functools

import jax
import jax.numpy as jnp
from jax import lax
from jax.experimental import pallas as pl
from jax.experimental.pallas import tpu as pltpu
from jax.experimental.pallas import tpu_sc as plsc

B = 4096
L = 200
EMBED = 64
EMBED_P = 128
HIDDEN = 128
CLASSES = 1000

NC = 2
NS = 16
NW = NC * NS
B_PER_W = B // NW
LANES = 16
NCH = EMBED // LANES

CHUNKS = ((0, 104), (104, 96))

_mesh = plsc.VectorSubcoreMesh(core_axis_name="c", subcore_axis_name="s")


@functools.partial(
    pl.kernel,
    out_type=jax.ShapeDtypeStruct((B, EMBED), jnp.float32),
    mesh=_mesh,
    scratch_types=[
        pltpu.VMEM((B_PER_W * L,), jnp.int32),
        pltpu.VMEM((L, EMBED_P), jnp.float32),
        pltpu.VMEM((L, EMBED_P), jnp.float32),
        pltpu.VMEM((B_PER_W, EMBED), jnp.float32),
        pltpu.SemaphoreType.DMA,
        pltpu.SemaphoreType.DMA,
    ],
)
def _pool_kernel(table_hbm, text_hbm, out_hbm, idx_v, buf_a, buf_b, pooled_v,
                 sem_a, sem_b):
    wid = lax.axis_index("s") * NC + lax.axis_index("c")
    base = wid * B_PER_W
    pltpu.sync_copy(text_hbm.at[pl.ds(base * L, B_PER_W * L)], idx_v)

    def issue(r, buf, sem):
        rbase = pl.multiple_of(r * L, 8)
        for off, size in CHUNKS:
            pltpu.async_copy(
                table_hbm.at[idx_v.at[pl.ds(rbase + off, size)]],
                buf.at[pl.ds(off, size), :],
                sem,
            )

    def drain(sem):
        pltpu.make_async_copy(
            table_hbm.at[pl.ds(0, L), :], buf_a, sem
        ).wait()

    def accumulate(r, buf):
        def acc_body(t, accs):
            return tuple(
                accs[c] + buf[t, pl.ds(c * LANES, LANES)] for c in range(NCH)
            )

        accs = tuple(jnp.zeros((LANES,), jnp.float32) for _ in range(NCH))
        accs = lax.fori_loop(0, L, acc_body, accs, unroll=8)
        scale = jnp.float32(1.0 / L)
        for c in range(NCH):
            pooled_v[r, pl.ds(c * LANES, LANES)] = accs[c] * scale

    issue(0, buf_a, sem_a)

    def pair_body(k, carry):
        ra = 2 * k
        rb = 2 * k + 1
        issue(rb, buf_b, sem_b)
        drain(sem_a)
        accumulate(ra, buf_a)

        @pl.when(k < B_PER_W // 2 - 1)
        def _():
            issue(ra + 2, buf_a, sem_a)

        drain(sem_b)
        accumulate(rb, buf_b)
        return carry

    lax.fori_loop(0, B_PER_W // 2, pair_body, 0)
    pltpu.sync_copy(pooled_v, out_hbm.at[pl.ds(base, B_PER_W), :])


VOCAB = 1000000
TCOLS = 16384


def _transpose_pad_body(xt_ref, o_ref):
    o_ref[:, :EMBED] = xt_ref[...].T
    o_ref[:, EMBED:] = jnp.zeros((TCOLS, EMBED_P - EMBED), jnp.float32)


def _transpose_pad(table_t):
    return pl.pallas_call(
        _transpose_pad_body,
        grid=(pl.cdiv(VOCAB, TCOLS),),
        in_specs=[pl.BlockSpec((EMBED, TCOLS), lambda i: (0, i))],
        out_specs=pl.BlockSpec((TCOLS, EMBED_P), lambda i: (i, 0)),
        out_shape=jax.ShapeDtypeStruct((VOCAB, EMBED_P), jnp.float32),
    )(table_t)


def _mlp_body(x_ref, fcw_ref, fcb_ref, outw_ref, outb_ref, o_ref):
    h = jnp.tanh(
        lax.dot_general(
            x_ref[...], fcw_ref[...], (((1,), (1,)), ((), ())),
            preferred_element_type=jnp.float32,
        )
        + fcb_ref[...]
    )
    o_ref[...] = (
        lax.dot_general(
            outw_ref[...], h, (((1,), (1,)), ((), ())),
            preferred_element_type=jnp.float32,
        )
        + outb_ref[...]
    )


def kernel(text, embed_table, fc_w, fc_b, out_w, out_b):
    table_p = _transpose_pad(embed_table.T)
    pooled = _pool_kernel(table_p, text.reshape(-1))

    bt = 512
    out_t = pl.pallas_call(
        _mlp_body,
        grid=(B // bt,),
        in_specs=[
            pl.BlockSpec((bt, EMBED), lambda i: (i, 0)),
            pl.BlockSpec((HIDDEN, EMBED), lambda i: (0, 0)),
            pl.BlockSpec((1, HIDDEN), lambda i: (0, 0)),
            pl.BlockSpec((CLASSES, HIDDEN), lambda i: (0, 0)),
            pl.BlockSpec((CLASSES, 1), lambda i: (0, 0)),
        ],
        out_specs=pl.BlockSpec((CLASSES, bt), lambda i: (0, i)),
        out_shape=jax.ShapeDtypeStruct((CLASSES, B), jnp.float32),
    )(pooled, fc_w, fc_b.reshape(1, HIDDEN), out_w, out_b.reshape(CLASSES, 1))
    return out_t.T


# speedup vs baseline: 1.6499x; 1.0680x over previous
import functools

import jax
import jax.numpy as jnp
from jax import lax
from jax.experimental import pallas as pl
from jax.experimental.pallas import tpu as pltpu
from jax.experimental.pallas import tpu_sc as plsc

B = 4096
L = 200
EMBED = 64
EMBED_P = 128
HIDDEN = 128
CLASSES = 1000

NC = 2
NS = 16
NW = NC * NS
B_PER_W = B // NW
LANES = 16
NCH = EMBED // LANES

CHUNKS = ((0, 104), (104, 96))

_mesh = plsc.VectorSubcoreMesh(core_axis_name="c", subcore_axis_name="s")


CH0, CH1 = CHUNKS[0][1], CHUNKS[1][1]


@functools.partial(
    pl.kernel,
    out_type=jax.ShapeDtypeStruct((B, EMBED), jnp.float32),
    mesh=_mesh,
    scratch_types=[
        pltpu.VMEM((B_PER_W * L,), jnp.int32),
        [pltpu.VMEM((CH0, EMBED_P), jnp.float32) for _ in range(6)],
        pltpu.VMEM((B_PER_W, EMBED), jnp.float32),
        [pltpu.SemaphoreType.DMA for _ in range(6)],
    ],
)
def _pool_kernel(table_hbm, text_hbm, out_hbm, idx_v, bufs, pooled_v, sems):
    wid = lax.axis_index("s") * NC + lax.axis_index("c")
    base = wid * B_PER_W
    pltpu.sync_copy(text_hbm.at[pl.ds(base * L, B_PER_W * L)], idx_v)

    def issue(r, slot):
        rbase = pl.multiple_of(r * L, 8)
        for j, (off, size) in enumerate(CHUNKS):
            pltpu.async_copy(
                table_hbm.at[idx_v.at[pl.ds(rbase + off, size)]],
                bufs[slot + j].at[pl.ds(0, size), :],
                sems[slot + j],
            )

    def drain(slot):
        for j, (_, size) in enumerate(CHUNKS):
            pltpu.make_async_copy(
                table_hbm.at[pl.ds(0, size), :],
                bufs[slot + j].at[pl.ds(0, size), :],
                sems[slot + j],
            ).wait()

    def accumulate(r, slot):
        def make_body(buf):
            def acc_body(t, accs):
                return tuple(
                    accs[c] + buf[t, pl.ds(c * LANES, LANES)]
                    for c in range(NCH)
                )
            return acc_body

        accs = tuple(jnp.zeros((LANES,), jnp.float32) for _ in range(NCH))
        accs = lax.fori_loop(0, CH0, make_body(bufs[slot]), accs, unroll=8)
        accs = lax.fori_loop(0, CH1, make_body(bufs[slot + 1]), accs, unroll=8)
        scale = jnp.float32(1.0 / L)
        for c in range(NCH):
            pooled_v[r, pl.ds(c * LANES, LANES)] = accs[c] * scale

    for r0 in range(3):
        issue(r0, 2 * r0)

    def triple_body(k, carry):
        for j in range(3):
            r = 3 * k + j
            drain(2 * j)
            accumulate(r, 2 * j)

            @pl.when(r + 3 < B_PER_W)
            def _():
                issue(r + 3, 2 * j)
        return carry

    lax.fori_loop(0, B_PER_W // 3, triple_body, 0)
    for r in (126, 127):
        j = r % 3
        drain(2 * j)
        accumulate(r, 2 * j)

    pltpu.sync_copy(pooled_v, out_hbm.at[pl.ds(base, B_PER_W), :])


VOCAB = 1000000
TCOLS = 16384


def _transpose_pad_body(xt_ref, o_ref):
    o_ref[:, :EMBED] = xt_ref[...].T
    o_ref[:, EMBED:] = jnp.zeros((TCOLS, EMBED_P - EMBED), jnp.float32)


def _transpose_pad(table_t):
    return pl.pallas_call(
        _transpose_pad_body,
        grid=(pl.cdiv(VOCAB, TCOLS),),
        in_specs=[pl.BlockSpec((EMBED, TCOLS), lambda i: (0, i))],
        out_specs=pl.BlockSpec((TCOLS, EMBED_P), lambda i: (i, 0)),
        out_shape=jax.ShapeDtypeStruct((VOCAB, EMBED_P), jnp.float32),
    )(table_t)


def _mlp_body(x_ref, fcw_ref, fcb_ref, outw_ref, outb_ref, o_ref):
    h = jnp.tanh(
        lax.dot_general(
            x_ref[...], fcw_ref[...], (((1,), (1,)), ((), ())),
            preferred_element_type=jnp.float32,
        )
        + fcb_ref[...]
    )
    o_ref[...] = (
        lax.dot_general(
            outw_ref[...], h, (((1,), (1,)), ((), ())),
            preferred_element_type=jnp.float32,
        )
        + outb_ref[...]
    )


def kernel(text, embed_table, fc_w, fc_b, out_w, out_b):
    table_p = _transpose_pad(embed_table.T)
    pooled = _pool_kernel(table_p, text.reshape(-1))

    bt = 512
    out_t = pl.pallas_call(
        _mlp_body,
        grid=(B // bt,),
        in_specs=[
            pl.BlockSpec((bt, EMBED), lambda i: (i, 0)),
            pl.BlockSpec((HIDDEN, EMBED), lambda i: (0, 0)),
            pl.BlockSpec((1, HIDDEN), lambda i: (0, 0)),
            pl.BlockSpec((CLASSES, HIDDEN), lambda i: (0, 0)),
            pl.BlockSpec((CLASSES, 1), lambda i: (0, 0)),
        ],
        out_specs=pl.BlockSpec((CLASSES, bt), lambda i: (0, i)),
        out_shape=jax.ShapeDtypeStruct((CLASSES, B), jnp.float32),
    )(pooled, fc_w, fc_b.reshape(1, HIDDEN), out_w, out_b.reshape(CLASSES, 1))
    return out_t.T


# speedup vs baseline: 1.6798x; 1.0181x over previous
import functools

import jax
import jax.numpy as jnp
from jax import lax
from jax.experimental import pallas as pl
from jax.experimental.pallas import tpu as pltpu
from jax.experimental.pallas import tpu_sc as plsc

B = 4096
L = 200
EMBED = 64
EMBED_P = 128
HIDDEN = 128
CLASSES = 1000

NC = 2
NS = 16
NW = NC * NS
B_PER_W = B // NW
LANES = 16
NCH = EMBED // LANES

CHUNKS = ((0, 104), (104, 96))

_mesh = plsc.VectorSubcoreMesh(core_axis_name="c", subcore_axis_name="s")


CH0, CH1 = CHUNKS[0][1], CHUNKS[1][1]


@functools.partial(
    pl.kernel,
    out_type=jax.ShapeDtypeStruct((B, EMBED), jnp.float32),
    mesh=_mesh,
    scratch_types=[
        pltpu.VMEM((B_PER_W * L,), jnp.int32),
        [pltpu.VMEM((CH0, EMBED_P), jnp.float32) for _ in range(6)],
        pltpu.VMEM((B_PER_W, EMBED), jnp.float32),
        [pltpu.SemaphoreType.DMA for _ in range(6)],
    ],
)
def _pool_kernel(table_hbm, text_hbm, out_hbm, idx_v, bufs, pooled_v, sems):
    wid = lax.axis_index("s") * NC + lax.axis_index("c")
    base = wid * B_PER_W
    pltpu.sync_copy(text_hbm.at[pl.ds(base * L, B_PER_W * L)], idx_v)

    def issue(r, slot):
        rbase = pl.multiple_of(r * L, 8)
        for j, (off, size) in enumerate(CHUNKS):
            pltpu.async_copy(
                table_hbm.at[idx_v.at[pl.ds(rbase + off, size)]],
                bufs[slot + j].at[pl.ds(0, size), :],
                sems[slot + j],
            )

    def drain(slot):
        for j, (_, size) in enumerate(CHUNKS):
            pltpu.make_async_copy(
                table_hbm.at[pl.ds(0, size), :],
                bufs[slot + j].at[pl.ds(0, size), :],
                sems[slot + j],
            ).wait()

    def accumulate(r, slot):
        def make_body(buf):
            def acc_body(t, accs):
                return tuple(
                    accs[c] + buf[t, pl.ds(c * LANES, LANES)]
                    for c in range(NCH)
                )
            return acc_body

        accs = tuple(jnp.zeros((LANES,), jnp.float32) for _ in range(NCH))
        accs = lax.fori_loop(0, CH0, make_body(bufs[slot]), accs, unroll=8)
        accs = lax.fori_loop(0, CH1, make_body(bufs[slot + 1]), accs, unroll=8)
        scale = jnp.float32(1.0 / L)
        for c in range(NCH):
            pooled_v[r, pl.ds(c * LANES, LANES)] = accs[c] * scale

    for r0 in range(3):
        issue(r0, 2 * r0)

    def triple_body(k, carry):
        for j in range(3):
            r = 3 * k + j
            drain(2 * j)
            accumulate(r, 2 * j)

            @pl.when(r + 3 < B_PER_W)
            def _():
                issue(r + 3, 2 * j)
        return carry

    lax.fori_loop(0, B_PER_W // 3, triple_body, 0)
    for r in (126, 127):
        j = r % 3
        drain(2 * j)
        accumulate(r, 2 * j)

    pltpu.sync_copy(pooled_v, out_hbm.at[pl.ds(base, B_PER_W), :])


VOCAB = 1000000
TCOLS = 32768


def _transpose_pad_body(xt_ref, o_ref):
    o_ref[:, :EMBED] = xt_ref[...].T
    o_ref[:, EMBED:] = jnp.zeros((TCOLS, EMBED_P - EMBED), jnp.float32)


def _transpose_pad(table_t):
    return pl.pallas_call(
        _transpose_pad_body,
        grid=(pl.cdiv(VOCAB, TCOLS),),
        in_specs=[pl.BlockSpec((EMBED, TCOLS), lambda i: (0, i))],
        out_specs=pl.BlockSpec((TCOLS, EMBED_P), lambda i: (i, 0)),
        out_shape=jax.ShapeDtypeStruct((VOCAB, EMBED_P), jnp.float32),
    )(table_t)


def _mlp_body(x_ref, fcw_ref, fcb_ref, outw_ref, outb_ref, o_ref):
    h = jnp.tanh(
        lax.dot_general(
            x_ref[...], fcw_ref[...], (((1,), (1,)), ((), ())),
            preferred_element_type=jnp.float32,
        )
        + fcb_ref[...]
    )
    o_ref[...] = (
        lax.dot_general(
            outw_ref[...], h, (((1,), (1,)), ((), ())),
            preferred_element_type=jnp.float32,
        )
        + outb_ref[...]
    )


def kernel(text, embed_table, fc_w, fc_b, out_w, out_b):
    table_p = _transpose_pad(embed_table.T)
    pooled = _pool_kernel(table_p, text.reshape(-1))

    bt = 1024
    out_t = pl.pallas_call(
        _mlp_body,
        grid=(B // bt,),
        in_specs=[
            pl.BlockSpec((bt, EMBED), lambda i: (i, 0)),
            pl.BlockSpec((HIDDEN, EMBED), lambda i: (0, 0)),
            pl.BlockSpec((1, HIDDEN), lambda i: (0, 0)),
            pl.BlockSpec((CLASSES, HIDDEN), lambda i: (0, 0)),
            pl.BlockSpec((CLASSES, 1), lambda i: (0, 0)),
        ],
        out_specs=pl.BlockSpec((CLASSES, bt), lambda i: (0, i)),
        out_shape=jax.ShapeDtypeStruct((CLASSES, B), jnp.float32),
    )(pooled, fc_w, fc_b.reshape(1, HIDDEN), out_w, out_b.reshape(CLASSES, 1))
    return out_t.T
